# baseline (speedup 1.0000x reference)
import functools

import jax
import jax.numpy as jnp
from jax import lax
from jax.experimental import pallas as pl
from jax.experimental.pallas import tpu as pltpu
from jax.experimental.pallas import tpu_sc as plsc

NN = 10000
EE = 160000
DIN = 256
DH = 256
DOUT = 128

NC = 2
NS = 16
ROWS_PER_TILE = 640
NPAD = NS * ROWS_PER_TILE
BB = 80
ES1 = 10240
ES2 = 5120
NB1 = ES1 // BB
NB2 = ES2 // BB

_mesh = plsc.VectorSubcoreMesh(core_axis_name="c", subcore_axis_name="s")


DCD = 128
NDB = ES2 // BB


@functools.partial(
    pl.kernel,
    out_type=jax.ShapeDtypeStruct((NC * NPAD, DCD), jnp.float32),
    mesh=_mesh,
    scratch_types=[
        pltpu.VMEM((BB,), jnp.int32),
        pltpu.VMEM((BB, DCD), jnp.float32),
        pltpu.VMEM_SHARED((NPAD, DCD), jnp.float32),
    ],
)
def _sc_degree(dst_hbm, ones_hbm, zrows_hbm, out_hbm, idx_d, ones_v, acc_sh):
    c = lax.axis_index("c")
    s = lax.axis_index("s")
    wid = c * NS + s

    row0 = pl.multiple_of(s * ROWS_PER_TILE, 8)
    pltpu.sync_copy(zrows_hbm, acc_sh.at[pl.ds(row0, ROWS_PER_TILE)])
    pltpu.sync_copy(ones_hbm, ones_v)
    plsc.subcore_barrier()

    ebase = wid * ES2

    def body(i, _):
        off = pl.multiple_of(ebase + i * BB, 8)
        pltpu.sync_copy(dst_hbm.at[pl.ds(off, BB)], idx_d)
        pltpu.sync_copy(ones_v, acc_sh.at[idx_d], add=True)
        return 0
    lax.fori_loop(0, NDB, body, 0)

    plsc.subcore_barrier()
    out0 = pl.multiple_of(c * NPAD + s * ROWS_PER_TILE, 8)
    pltpu.sync_copy(acc_sh.at[pl.ds(row0, ROWS_PER_TILE)],
                    out_hbm.at[pl.ds(out0, ROWS_PER_TILE)])


def _make_sc_agg(dc, esub, nb):
    @functools.partial(
        pl.kernel,
        out_type=jax.ShapeDtypeStruct((NC * NPAD, dc), jnp.float32),
        mesh=_mesh,
        scratch_types=[
            pltpu.VMEM((BB,), jnp.int32),
            pltpu.VMEM((BB,), jnp.int32),
            pltpu.VMEM((BB,), jnp.int32),
            pltpu.VMEM((BB,), jnp.int32),
            pltpu.VMEM((BB, dc), jnp.float32),
            pltpu.VMEM((BB, dc), jnp.float32),
            pltpu.VMEM_SHARED((NPAD, dc), jnp.float32),
            pltpu.SemaphoreType.DMA,
            pltpu.SemaphoreType.DMA,
        ],
    )
    def agg(table_hbm, src_hbm, dst_hbm, zrows_hbm, out_hbm,
            idx_s0, idx_s1, idx_d0, idx_d1, rows0, rows1, acc_sh,
            sg0, sg1):
        c = lax.axis_index("c")
        s = lax.axis_index("s")

        row0 = pl.multiple_of(s * ROWS_PER_TILE, 8)
        pltpu.sync_copy(zrows_hbm, acc_sh.at[pl.ds(row0, ROWS_PER_TILE)])
        plsc.subcore_barrier()

        ebase = (c * NS + s) * esub
        idx_s = (idx_s0, idx_s1)
        idx_d = (idx_d0, idx_d1)
        rows = (rows0, rows1)
        sg = (sg0, sg1)

        def body(i, _):
            off = pl.multiple_of(ebase + i * BB, 8)
            pltpu.sync_copy(src_hbm.at[pl.ds(off, BB)], idx_s[0])
            pltpu.sync_copy(dst_hbm.at[pl.ds(off, BB)], idx_d[0])
            pltpu.async_copy(table_hbm.at[idx_s[0]], rows[0], sg[0]).wait()
            pltpu.sync_copy(rows[0], acc_sh.at[idx_d[0]], add=True)
            return 0
        lax.fori_loop(0, nb, body, 0)

        plsc.subcore_barrier()
        out0 = pl.multiple_of(c * NPAD + s * ROWS_PER_TILE, 8)
        pltpu.sync_copy(acc_sh.at[pl.ds(row0, ROWS_PER_TILE)],
                        out_hbm.at[pl.ds(out0, ROWS_PER_TILE)])

    return agg


_sc_agg_l1 = _make_sc_agg(DH // 2, ES1, NB1)
_sc_agg_l2 = _make_sc_agg(DOUT, ES2, NB2)


def _tc_dinv_body(part_ref, out_ref):
    deg = jnp.sum(part_ref[...], axis=0)
    deg = deg[:NN, 0:1] + 1.0
    out_ref[...] = lax.rsqrt(deg)


def _tc_dinv(part):
    return pl.pallas_call(
        _tc_dinv_body,
        out_shape=jax.ShapeDtypeStruct((NN, 1), jnp.float32),
    )(part)


_NB = 10
_NBR = NN // _NB


def _tc_mm1_body(x_ref, w_ref, o_ref):
    o_ref[...] = jnp.dot(x_ref[...], w_ref[...],
                         preferred_element_type=jnp.float32)[None]


def _tc_mm1(x, W1):
    return pl.pallas_call(
        _tc_mm1_body,
        grid=(NC, _NB),
        in_specs=[
            pl.BlockSpec((_NBR, DIN), lambda c, i: (i, 0)),
            pl.BlockSpec((DIN, DH // 2), lambda c, i: (0, c)),
        ],
        out_specs=pl.BlockSpec((1, _NBR, DH // 2), lambda c, i: (c, i, 0)),
        out_shape=jax.ShapeDtypeStruct((NC, NN, DH // 2), jnp.float32),
    )(x, W1)


def _tc_scale_body(h_ref, d_ref, o_ref):
    o_ref[...] = h_ref[...] * d_ref[...][None]


def _tc_scale(h, dinv, dc):
    return pl.pallas_call(
        _tc_scale_body,
        grid=(NC, _NB),
        in_specs=[
            pl.BlockSpec((1, _NBR, dc), lambda c, i: (c, i, 0)),
            pl.BlockSpec((_NBR, 1), lambda c, i: (i, 0)),
        ],
        out_specs=pl.BlockSpec((1, _NBR, dc), lambda c, i: (c, i, 0)),
        out_shape=jax.ShapeDtypeStruct((NC, NN, dc), jnp.float32),
    )(h, dinv)


def _tc_z_body(s_ref, hs_ref, d_ref, b_ref, z_ref, ps_ref, pq_ref):
    i = pl.program_id(1)
    z = d_ref[...] * (s_ref[0] + hs_ref[0]) + b_ref[...][None, :]
    z_ref[...] = z[None]
    zpad = jnp.zeros((7, DH // 2), jnp.float32)
    part = jnp.concatenate([jnp.sum(z, axis=0)[None], zpad], axis=0)[None]
    partq = jnp.concatenate([jnp.sum(z * z, axis=0)[None], zpad], axis=0)[None]

    @pl.when(i == 0)
    def _():
        ps_ref[...] = part
        pq_ref[...] = partq

    @pl.when(i != 0)
    def _():
        ps_ref[...] += part
        pq_ref[...] += partq


def _tc_z_stats(S1, hs, dinv, b1):
    return pl.pallas_call(
        _tc_z_body,
        grid=(NC, _NB),
        in_specs=[
            pl.BlockSpec((1, _NBR, DH // 2), lambda c, i: (c, i, 0)),
            pl.BlockSpec((1, _NBR, DH // 2), lambda c, i: (c, i, 0)),
            pl.BlockSpec((_NBR, 1), lambda c, i: (i, 0)),
            pl.BlockSpec((DH // 2,), lambda c, i: (c,)),
        ],
        out_specs=[
            pl.BlockSpec((1, _NBR, DH // 2), lambda c, i: (c, i, 0)),
            pl.BlockSpec((1, 8, DH // 2), lambda c, i: (c, 0, 0)),
            pl.BlockSpec((1, 8, DH // 2), lambda c, i: (c, 0, 0)),
        ],
        out_shape=[
            jax.ShapeDtypeStruct((NC, NN, DH // 2), jnp.float32),
            jax.ShapeDtypeStruct((NC, 8, DH // 2), jnp.float32),
            jax.ShapeDtypeStruct((NC, 8, DH // 2), jnp.float32),
        ],
    )(S1, hs, dinv, b1)


def _tc_bn_mm2_body(z_ref, ps_ref, pq_ref, g_ref, be_ref, w_ref, d_ref, o_ref):
    acc = None
    for half in range(2):
        mean = jnp.sum(ps_ref[half], axis=0) / NN
        var = jnp.sum(pq_ref[half], axis=0) / NN - mean * mean
        scale = lax.rsqrt(var + 1e-5) * g_ref[pl.ds(half * (DH // 2), DH // 2)]
        shift = be_ref[pl.ds(half * (DH // 2), DH // 2)] - mean * scale
        hbn = jnp.maximum(z_ref[half] * scale[None, :] + shift[None, :], 0.0)
        part = jnp.dot(hbn, w_ref[pl.ds(half * (DH // 2), DH // 2), :],
                       preferred_element_type=jnp.float32)
        acc = part if acc is None else acc + part
    o_ref[...] = acc * d_ref[...]


def _tc_bn_mm2(z, ps, pq, gamma, beta, W2, dinv):
    return pl.pallas_call(
        _tc_bn_mm2_body,
        grid=(_NB,),
        in_specs=[
            pl.BlockSpec((NC, _NBR, DH // 2), lambda i: (0, i, 0)),
            pl.BlockSpec((NC, 8, DH // 2), lambda i: (0, 0, 0)),
            pl.BlockSpec((NC, 8, DH // 2), lambda i: (0, 0, 0)),
            pl.BlockSpec((DH,), lambda i: (0,)),
            pl.BlockSpec((DH,), lambda i: (0,)),
            pl.BlockSpec((DH, DOUT), lambda i: (0, 0)),
            pl.BlockSpec((_NBR, 1), lambda i: (i, 0)),
        ],
        out_specs=pl.BlockSpec((_NBR, DOUT), lambda i: (i, 0)),
        out_shape=jax.ShapeDtypeStruct((NN, DOUT), jnp.float32),
    )(z, ps, pq, gamma, beta, W2, dinv)


def _tc_final_body(s_ref, hs_ref, d_ref, b_ref, o_ref):
    o = (d_ref[...] * (s_ref[0] + s_ref[1] + hs_ref[...])
         + b_ref[...][None, :])
    m = jnp.max(o, axis=1, keepdims=True)
    e = jnp.exp(o - m)
    ssum = jnp.sum(e, axis=1, keepdims=True)
    o_ref[...] = o - m - jnp.log(ssum)


def _tc_final(S2, hs2, dinv, b2):
    return pl.pallas_call(
        _tc_final_body,
        grid=(_NB,),
        in_specs=[
            pl.BlockSpec((NC, _NBR, DOUT), lambda i: (0, i, 0)),
            pl.BlockSpec((_NBR, DOUT), lambda i: (i, 0)),
            pl.BlockSpec((_NBR, 1), lambda i: (i, 0)),
            pl.BlockSpec((DOUT,), lambda i: (0,)),
        ],
        out_specs=pl.BlockSpec((_NBR, DOUT), lambda i: (i, 0)),
        out_shape=jax.ShapeDtypeStruct((NN, DOUT), jnp.float32),
    )(S2, hs2, dinv, b2)


def kernel(x, edge_index, W1, b1, gamma, beta, W2, b2):
    src = edge_index[0].astype(jnp.int32)
    dst = edge_index[1].astype(jnp.int32)

    src1 = jnp.pad(src.reshape(NS, EE // NS),
                   ((0, 0), (0, ES1 - EE // NS)))
    dst1 = jnp.pad(dst.reshape(NS, EE // NS),
                   ((0, 0), (0, ES1 - EE // NS)), constant_values=NN)
    srcA = jnp.stack([src1, src1 + NN]).reshape(-1)
    dstA = jnp.stack([dst1, dst1]).reshape(-1)

    srcB = jnp.pad(src.reshape(NC * NS, EE // (NC * NS)),
                   ((0, 0), (0, ES2 - EE // (NC * NS)))).reshape(-1)
    dstB = jnp.pad(dst.reshape(NC * NS, EE // (NC * NS)),
                   ((0, 0), (0, ES2 - EE // (NC * NS))),
                   constant_values=NN).reshape(-1)

    zrows128 = jnp.zeros((ROWS_PER_TILE, DH // 2), jnp.float32)
    ones128 = jnp.ones((BB, DCD), jnp.float32)

    deg_part = _sc_degree(dstB, ones128, zrows128)
    dinv = _tc_dinv(deg_part.reshape(NC, NPAD, DCD))

    h = _tc_mm1(x, W1)
    hs = _tc_scale(h, dinv, DH // 2)

    S1 = _sc_agg_l1(hs.reshape(NC * NN, DH // 2), srcA, dstA, zrows128)
    S1 = S1.reshape(NC, NPAD, DH // 2)

    z, ps, pq = _tc_z_stats(S1, hs, dinv, b1)
    hs2 = _tc_bn_mm2(z, ps, pq, gamma, beta, W2, dinv)

    S2 = _sc_agg_l2(hs2, srcB, dstB, zrows128)
    S2 = S2.reshape(NC, NPAD, DOUT)

    return _tc_final(S2, hs2, dinv, b2)


# speedup vs baseline: 1.2432x; 1.2432x over previous
import functools

import jax
import jax.numpy as jnp
from jax import lax
from jax.experimental import pallas as pl
from jax.experimental.pallas import tpu as pltpu
from jax.experimental.pallas import tpu_sc as plsc

NN = 10000
EE = 160000
DIN = 256
DH = 256
DOUT = 128

NC = 2
NS = 16
ROWS_PER_TILE = 640
NPAD = NS * ROWS_PER_TILE
BB = 80
ES1 = 10240
ES2 = 5120
NB1 = ES1 // BB
NB2 = ES2 // BB

_mesh = plsc.VectorSubcoreMesh(core_axis_name="c", subcore_axis_name="s")


DCD = 128
NDB = ES2 // BB


@functools.partial(
    pl.kernel,
    out_type=jax.ShapeDtypeStruct((NC * NPAD, DCD), jnp.float32),
    mesh=_mesh,
    scratch_types=[
        pltpu.VMEM((BB,), jnp.int32),
        pltpu.VMEM((BB, DCD), jnp.float32),
        pltpu.VMEM_SHARED((NPAD, DCD), jnp.float32),
    ],
)
def _sc_degree(dst_hbm, ones_hbm, zrows_hbm, out_hbm, idx_d, ones_v, acc_sh):
    c = lax.axis_index("c")
    s = lax.axis_index("s")
    wid = c * NS + s

    row0 = pl.multiple_of(s * ROWS_PER_TILE, 8)
    pltpu.sync_copy(zrows_hbm, acc_sh.at[pl.ds(row0, ROWS_PER_TILE)])
    pltpu.sync_copy(ones_hbm, ones_v)
    plsc.subcore_barrier()

    ebase = wid * ES2

    def body(i, _):
        off = pl.multiple_of(ebase + i * BB, 8)
        pltpu.sync_copy(dst_hbm.at[pl.ds(off, BB)], idx_d)
        pltpu.sync_copy(ones_v, acc_sh.at[idx_d], add=True)
        return 0
    lax.fori_loop(0, NDB, body, 0)

    plsc.subcore_barrier()
    out0 = pl.multiple_of(c * NPAD + s * ROWS_PER_TILE, 8)
    pltpu.sync_copy(acc_sh.at[pl.ds(row0, ROWS_PER_TILE)],
                    out_hbm.at[pl.ds(out0, ROWS_PER_TILE)])


def _make_sc_agg(dc, esub, nb):
    @functools.partial(
        pl.kernel,
        out_type=jax.ShapeDtypeStruct((NC * NPAD, dc), jnp.float32),
        mesh=_mesh,
        scratch_types=[
            pltpu.VMEM((BB,), jnp.int32),
            pltpu.VMEM((BB,), jnp.int32),
            pltpu.VMEM((BB,), jnp.int32),
            pltpu.VMEM((BB,), jnp.int32),
            pltpu.VMEM((BB, dc), jnp.float32),
            pltpu.VMEM((BB, dc), jnp.float32),
            pltpu.VMEM_SHARED((NPAD, dc), jnp.float32),
            pltpu.SemaphoreType.DMA,
            pltpu.SemaphoreType.DMA,
        ],
    )
    def agg(table_hbm, src_hbm, dst_hbm, zrows_hbm, out_hbm,
            idx_s0, idx_s1, idx_d0, idx_d1, rows0, rows1, acc_sh,
            sg0, sg1):
        c = lax.axis_index("c")
        s = lax.axis_index("s")

        row0 = pl.multiple_of(s * ROWS_PER_TILE, 8)
        pltpu.sync_copy(zrows_hbm, acc_sh.at[pl.ds(row0, ROWS_PER_TILE)])
        plsc.subcore_barrier()

        ebase = (c * NS + s) * esub
        idx_s = (idx_s0, idx_s1)
        idx_d = (idx_d0, idx_d1)
        rows = (rows0, rows1)
        sg = (sg0, sg1)

        def load(i, b):
            off = pl.multiple_of(ebase + i * BB, 8)
            pltpu.sync_copy(src_hbm.at[pl.ds(off, BB)], idx_s[b])
            pltpu.sync_copy(dst_hbm.at[pl.ds(off, BB)], idx_d[b])

        def start_g(b):
            pltpu.async_copy(table_hbm.at[idx_s[b]], rows[b], sg[b])

        def wait_g(b):
            pltpu.make_async_copy(table_hbm.at[idx_s[b]], rows[b],
                                  sg[b]).wait()

        def scat(b):
            pltpu.sync_copy(rows[b], acc_sh.at[idx_d[b]], add=True)

        load(0, 0)
        start_g(0)
        load(1, 1)
        start_g(1)

        def body(j, _):
            i0 = j * 2
            wait_g(0)
            scat(0)
            load(i0 + 2, 0)
            start_g(0)
            wait_g(1)
            scat(1)
            load(i0 + 3, 1)
            start_g(1)
            return 0
        lax.fori_loop(0, nb // 2 - 1, body, 0)

        wait_g(0)
        scat(0)
        wait_g(1)
        scat(1)

        plsc.subcore_barrier()
        out0 = pl.multiple_of(c * NPAD + s * ROWS_PER_TILE, 8)
        pltpu.sync_copy(acc_sh.at[pl.ds(row0, ROWS_PER_TILE)],
                        out_hbm.at[pl.ds(out0, ROWS_PER_TILE)])

    return agg


_sc_agg_l1 = _make_sc_agg(DH // 2, ES1, NB1)
_sc_agg_l2 = _make_sc_agg(DOUT, ES2, NB2)


def _tc_dinv_body(part_ref, out_ref):
    deg = jnp.sum(part_ref[...], axis=0)
    deg = deg[:NN, 0:1] + 1.0
    out_ref[...] = lax.rsqrt(deg)


def _tc_dinv(part):
    return pl.pallas_call(
        _tc_dinv_body,
        out_shape=jax.ShapeDtypeStruct((NN, 1), jnp.float32),
    )(part)


_NB = 10
_NBR = NN // _NB


def _tc_mm1_body(x_ref, w_ref, o_ref):
    o_ref[...] = jnp.dot(x_ref[...], w_ref[...],
                         preferred_element_type=jnp.float32)[None]


def _tc_mm1(x, W1):
    return pl.pallas_call(
        _tc_mm1_body,
        grid=(NC, _NB),
        in_specs=[
            pl.BlockSpec((_NBR, DIN), lambda c, i: (i, 0)),
            pl.BlockSpec((DIN, DH // 2), lambda c, i: (0, c)),
        ],
        out_specs=pl.BlockSpec((1, _NBR, DH // 2), lambda c, i: (c, i, 0)),
        out_shape=jax.ShapeDtypeStruct((NC, NN, DH // 2), jnp.float32),
    )(x, W1)


def _tc_scale_body(h_ref, d_ref, o_ref):
    o_ref[...] = h_ref[...] * d_ref[...][None]


def _tc_scale(h, dinv, dc):
    return pl.pallas_call(
        _tc_scale_body,
        grid=(NC, _NB),
        in_specs=[
            pl.BlockSpec((1, _NBR, dc), lambda c, i: (c, i, 0)),
            pl.BlockSpec((_NBR, 1), lambda c, i: (i, 0)),
        ],
        out_specs=pl.BlockSpec((1, _NBR, dc), lambda c, i: (c, i, 0)),
        out_shape=jax.ShapeDtypeStruct((NC, NN, dc), jnp.float32),
    )(h, dinv)


def _tc_z_body(s_ref, hs_ref, d_ref, b_ref, z_ref, ps_ref, pq_ref):
    i = pl.program_id(1)
    z = d_ref[...] * (s_ref[0] + hs_ref[0]) + b_ref[...][None, :]
    z_ref[...] = z[None]
    zpad = jnp.zeros((7, DH // 2), jnp.float32)
    part = jnp.concatenate([jnp.sum(z, axis=0)[None], zpad], axis=0)[None]
    partq = jnp.concatenate([jnp.sum(z * z, axis=0)[None], zpad], axis=0)[None]

    @pl.when(i == 0)
    def _():
        ps_ref[...] = part
        pq_ref[...] = partq

    @pl.when(i != 0)
    def _():
        ps_ref[...] += part
        pq_ref[...] += partq


def _tc_z_stats(S1, hs, dinv, b1):
    return pl.pallas_call(
        _tc_z_body,
        grid=(NC, _NB),
        in_specs=[
            pl.BlockSpec((1, _NBR, DH // 2), lambda c, i: (c, i, 0)),
            pl.BlockSpec((1, _NBR, DH // 2), lambda c, i: (c, i, 0)),
            pl.BlockSpec((_NBR, 1), lambda c, i: (i, 0)),
            pl.BlockSpec((DH // 2,), lambda c, i: (c,)),
        ],
        out_specs=[
            pl.BlockSpec((1, _NBR, DH // 2), lambda c, i: (c, i, 0)),
            pl.BlockSpec((1, 8, DH // 2), lambda c, i: (c, 0, 0)),
            pl.BlockSpec((1, 8, DH // 2), lambda c, i: (c, 0, 0)),
        ],
        out_shape=[
            jax.ShapeDtypeStruct((NC, NN, DH // 2), jnp.float32),
            jax.ShapeDtypeStruct((NC, 8, DH // 2), jnp.float32),
            jax.ShapeDtypeStruct((NC, 8, DH // 2), jnp.float32),
        ],
    )(S1, hs, dinv, b1)


def _tc_bn_mm2_body(z_ref, ps_ref, pq_ref, g_ref, be_ref, w_ref, d_ref, o_ref):
    acc = None
    for half in range(2):
        mean = jnp.sum(ps_ref[half], axis=0) / NN
        var = jnp.sum(pq_ref[half], axis=0) / NN - mean * mean
        scale = lax.rsqrt(var + 1e-5) * g_ref[pl.ds(half * (DH // 2), DH // 2)]
        shift = be_ref[pl.ds(half * (DH // 2), DH // 2)] - mean * scale
        hbn = jnp.maximum(z_ref[half] * scale[None, :] + shift[None, :], 0.0)
        part = jnp.dot(hbn, w_ref[pl.ds(half * (DH // 2), DH // 2), :],
                       preferred_element_type=jnp.float32)
        acc = part if acc is None else acc + part
    o_ref[...] = acc * d_ref[...]


def _tc_bn_mm2(z, ps, pq, gamma, beta, W2, dinv):
    return pl.pallas_call(
        _tc_bn_mm2_body,
        grid=(_NB,),
        in_specs=[
            pl.BlockSpec((NC, _NBR, DH // 2), lambda i: (0, i, 0)),
            pl.BlockSpec((NC, 8, DH // 2), lambda i: (0, 0, 0)),
            pl.BlockSpec((NC, 8, DH // 2), lambda i: (0, 0, 0)),
            pl.BlockSpec((DH,), lambda i: (0,)),
            pl.BlockSpec((DH,), lambda i: (0,)),
            pl.BlockSpec((DH, DOUT), lambda i: (0, 0)),
            pl.BlockSpec((_NBR, 1), lambda i: (i, 0)),
        ],
        out_specs=pl.BlockSpec((_NBR, DOUT), lambda i: (i, 0)),
        out_shape=jax.ShapeDtypeStruct((NN, DOUT), jnp.float32),
    )(z, ps, pq, gamma, beta, W2, dinv)


def _tc_final_body(s_ref, hs_ref, d_ref, b_ref, o_ref):
    o = (d_ref[...] * (s_ref[0] + s_ref[1] + hs_ref[...])
         + b_ref[...][None, :])
    m = jnp.max(o, axis=1, keepdims=True)
    e = jnp.exp(o - m)
    ssum = jnp.sum(e, axis=1, keepdims=True)
    o_ref[...] = o - m - jnp.log(ssum)


def _tc_final(S2, hs2, dinv, b2):
    return pl.pallas_call(
        _tc_final_body,
        grid=(_NB,),
        in_specs=[
            pl.BlockSpec((NC, _NBR, DOUT), lambda i: (0, i, 0)),
            pl.BlockSpec((_NBR, DOUT), lambda i: (i, 0)),
            pl.BlockSpec((_NBR, 1), lambda i: (i, 0)),
            pl.BlockSpec((DOUT,), lambda i: (0,)),
        ],
        out_specs=pl.BlockSpec((_NBR, DOUT), lambda i: (i, 0)),
        out_shape=jax.ShapeDtypeStruct((NN, DOUT), jnp.float32),
    )(S2, hs2, dinv, b2)


def kernel(x, edge_index, W1, b1, gamma, beta, W2, b2):
    src = edge_index[0].astype(jnp.int32)
    dst = edge_index[1].astype(jnp.int32)

    src1 = jnp.pad(src.reshape(NS, EE // NS),
                   ((0, 0), (0, ES1 - EE // NS)))
    dst1 = jnp.pad(dst.reshape(NS, EE // NS),
                   ((0, 0), (0, ES1 - EE // NS)), constant_values=NN)
    srcA = jnp.stack([src1, src1 + NN]).reshape(-1)
    dstA = jnp.stack([dst1, dst1]).reshape(-1)

    srcB = jnp.pad(src.reshape(NC * NS, EE // (NC * NS)),
                   ((0, 0), (0, ES2 - EE // (NC * NS)))).reshape(-1)
    dstB = jnp.pad(dst.reshape(NC * NS, EE // (NC * NS)),
                   ((0, 0), (0, ES2 - EE // (NC * NS))),
                   constant_values=NN).reshape(-1)

    zrows128 = jnp.zeros((ROWS_PER_TILE, DH // 2), jnp.float32)
    ones128 = jnp.ones((BB, DCD), jnp.float32)

    deg_part = _sc_degree(dstB, ones128, zrows128)
    dinv = _tc_dinv(deg_part.reshape(NC, NPAD, DCD))

    h = _tc_mm1(x, W1)
    hs = _tc_scale(h, dinv, DH // 2)

    S1 = _sc_agg_l1(hs.reshape(NC * NN, DH // 2), srcA, dstA, zrows128)
    S1 = S1.reshape(NC, NPAD, DH // 2)

    z, ps, pq = _tc_z_stats(S1, hs, dinv, b1)
    hs2 = _tc_bn_mm2(z, ps, pq, gamma, beta, W2, dinv)

    S2 = _sc_agg_l2(hs2, srcB, dstB, zrows128)
    S2 = S2.reshape(NC, NPAD, DOUT)

    return _tc_final(S2, hs2, dinv, b2)
